# Initial kernel scaffold; baseline (speedup 1.0000x reference)
import jax
import jax.numpy as jnp
from jax.experimental import pallas as pl


def kernel(x, pe, edge_index, edge_attr, batch, Wq0, bq0, Wk0, bk0, Wv0, bv0, We0, Ws0, bs0, Wq1, bq1, Wk1, bk1, Wv1, bv1, We1, Ws1, bs1, mlp_W1, mlp_b1, mlp_W2, mlp_b2):
    raise NotImplementedError("write your pallas kernel here")


# speedup vs baseline: 7.8535x; 7.8535x over previous
import functools

import jax
import jax.numpy as jnp
from jax import lax
from jax.experimental import pallas as pl
from jax.experimental.pallas import tpu as pltpu
from jax.experimental.pallas import tpu_sc as plsc

N = 10000
E = 320000
D = 128
ED = 16
C = 128

NPAD = 10112
BLK = 128
GRID = NPAD // BLK

NC = 2
NS = 16
NW = NC * NS
EPW = E // NW
CH = 80
NCHUNK = EPW // CH
ROWS_PER_S = NPAD // NS

_INV_SQRT_C = 1.0 / (C ** 0.5)


def _mm_body(a_ref, w_ref, b_ref, o_ref, *, act):
    o = jnp.dot(a_ref[...], w_ref[...], preferred_element_type=jnp.float32)
    o = o + b_ref[...]
    if act:
        o = jnp.where(o > 0, o, 0.01 * o)
    o_ref[...] = o


def _mm(a, w, b, act):
    m, k = a.shape
    o = w.shape[1]
    return pl.pallas_call(
        functools.partial(_mm_body, act=act),
        grid=(m // BLK,),
        in_specs=[
            pl.BlockSpec((BLK, k), lambda i: (i, 0)),
            pl.BlockSpec((k, o), lambda i: (0, 0)),
            pl.BlockSpec((1, o), lambda i: (0, 0)),
        ],
        out_specs=pl.BlockSpec((BLK, o), lambda i: (i, 0)),
        out_shape=jax.ShapeDtypeStruct((m, o), jnp.float32),
    )(a, w, b.reshape(1, o))


def _proj_body(x_ref, w_ref, b_ref, wet_ref, q_ref, k_ref, v_ref, s_ref, qe_ref):
    o = jnp.dot(x_ref[...], w_ref[...], preferred_element_type=jnp.float32)
    o = o + b_ref[...]
    q = o[:, 0:C]
    q_ref[...] = q
    k_ref[...] = o[:, C:2 * C]
    v_ref[...] = o[:, 2 * C:3 * C]
    s_ref[...] = o[:, 3 * C:4 * C]
    qe_ref[...] = jnp.dot(q, wet_ref[...], preferred_element_type=jnp.float32)


def _proj(x, wcat, bcat, wet):
    return pl.pallas_call(
        _proj_body,
        grid=(GRID,),
        in_specs=[
            pl.BlockSpec((BLK, D), lambda i: (i, 0)),
            pl.BlockSpec((D, 4 * C), lambda i: (0, 0)),
            pl.BlockSpec((1, 4 * C), lambda i: (0, 0)),
            pl.BlockSpec((C, ED), lambda i: (0, 0)),
        ],
        out_specs=[
            pl.BlockSpec((BLK, C), lambda i: (i, 0)),
            pl.BlockSpec((BLK, C), lambda i: (i, 0)),
            pl.BlockSpec((BLK, C), lambda i: (i, 0)),
            pl.BlockSpec((BLK, C), lambda i: (i, 0)),
            pl.BlockSpec((BLK, ED), lambda i: (i, 0)),
        ],
        out_shape=[
            jax.ShapeDtypeStruct((NPAD, C), jnp.float32),
            jax.ShapeDtypeStruct((NPAD, C), jnp.float32),
            jax.ShapeDtypeStruct((NPAD, C), jnp.float32),
            jax.ShapeDtypeStruct((NPAD, C), jnp.float32),
            jax.ShapeDtypeStruct((NPAD, ED), jnp.float32),
        ],
    )(x, wcat, bcat.reshape(1, 4 * C), wet)


def _combine_body(a128_ref, ae_ref, den_ref, skip_ref, we_ref, ones_ref, o_ref):
    a = a128_ref[0] + a128_ref[1]
    ae = ae_ref[0] + ae_ref[1]
    numer = a + jnp.dot(ae, we_ref[...], preferred_element_type=jnp.float32)
    den = lax.dot_general(den_ref[...], ones_ref[...],
                          (((0,), (0,)), ((), ())),
                          preferred_element_type=jnp.float32)
    o = numer / (den + 1e-16) + skip_ref[...]
    o_ref[...] = jnp.where(o > 0, o, 0.01 * o)


def _combine(a128, ae, den, skip, we):
    return pl.pallas_call(
        _combine_body,
        grid=(GRID,),
        in_specs=[
            pl.BlockSpec((2, BLK, C), lambda i: (0, i, 0)),
            pl.BlockSpec((2, BLK, ED), lambda i: (0, i, 0)),
            pl.BlockSpec((NW, BLK), lambda i: (0, i)),
            pl.BlockSpec((BLK, C), lambda i: (i, 0)),
            pl.BlockSpec((ED, C), lambda i: (0, 0)),
            pl.BlockSpec((NW, 1), lambda i: (0, 0)),
        ],
        out_specs=pl.BlockSpec((BLK, C), lambda i: (i, 0)),
        out_shape=jax.ShapeDtypeStruct((NPAD, C), jnp.float32),
    )(a128, ae, den, skip, we, jnp.ones((NW, 1), jnp.float32))


def _edge_body(src_hbm, dst_hbm, ea_hbm, q_hbm, k_hbm, v_hbm, qe_hbm,
               z128_hbm, zae_hbm, zden_hbm,
               a128_out, ae_out, den_out,
               sidx, didx, qvrows, krows, qerows, earows, eam,
               pbuf, exbuf, den, a128_sh, ae_sh,
               sem1, sem2, sem3, sem4):
    cid = lax.axis_index("c")
    sid = lax.axis_index("s")
    wid = sid * NC + cid
    tile_base = wid * EPW
    srow = sid * (N // NS)

    pltpu.sync_copy(z128_hbm.at[pl.ds(srow, N // NS)],
                    a128_sh.at[pl.ds(srow, N // NS)])
    pltpu.sync_copy(zae_hbm.at[pl.ds(srow, N // NS)],
                    ae_sh.at[pl.ds(srow, N // NS)])
    pltpu.sync_copy(zden_hbm, den)

    plsc.subcore_barrier()

    iota16 = lax.iota(jnp.int32, 16)

    def _chunk(j, _):
        base = tile_base + j * CH
        pltpu.sync_copy(src_hbm.at[pl.ds(base, CH)], sidx)
        pltpu.sync_copy(dst_hbm.at[pl.ds(base, CH)], didx)
        cp_q = pltpu.async_copy(q_hbm.at[didx], qvrows, sem1)
        cp_k = pltpu.async_copy(k_hbm.at[sidx], krows, sem2)
        cp_qe = pltpu.async_copy(qe_hbm.at[didx], qerows, sem4)
        pltpu.sync_copy(ea_hbm.at[pl.ds(base, CH)], earows)
        cp_q.wait()
        cp_k.wait()
        cp_qe.wait()

        def _group(g, _):
            e0 = g * 16
            for ii in range(16):
                e = e0 + ii
                acc = qerows[e, :] * earows[e, :]
                for c in range(C // 16):
                    acc = acc + (qvrows[e, pl.ds(c * 16, 16)] *
                                 krows[e, pl.ds(c * 16, 16)])
                pbuf[ii, pl.ds(0, 16)] = acc
            tot = plsc.load_gather(pbuf, [iota16, jnp.zeros((16,), jnp.int32)])
            for c in range(1, 16):
                tot = tot + plsc.load_gather(
                    pbuf, [iota16, jnp.full((16,), c, jnp.int32)])
            ex = jnp.exp(tot * _INV_SQRT_C)
            exbuf[pl.ds(e0, 16)] = ex
            didx_v = didx[pl.ds(e0, 16)]
            plsc.addupdate_scatter(den, [didx_v], ex)
            return 0
        lax.fori_loop(0, CH // 16, _group, 0)

        cp_v = pltpu.async_copy(v_hbm.at[sidx], qvrows, sem3)

        def _eascale(e, _):
            exs = plsc.load_gather(exbuf, [jnp.full((16,), e, jnp.int32)])
            eam[e, :] = earows[e, :] * exs
            return 0
        lax.fori_loop(0, CH, _eascale, 0)

        cp_v.wait()

        def _vscale(e, _):
            exs = plsc.load_gather(exbuf, [jnp.full((16,), e, jnp.int32)])
            for c in range(C // 16):
                qvrows[e, pl.ds(c * 16, 16)] = (
                    qvrows[e, pl.ds(c * 16, 16)] * exs)
            return 0
        lax.fori_loop(0, CH, _vscale, 0)

        pltpu.sync_copy(qvrows, a128_sh.at[didx], add=True)
        pltpu.sync_copy(eam, ae_sh.at[didx], add=True)
        return 0

    lax.fori_loop(0, NCHUNK, _chunk, 0)

    pltpu.sync_copy(den, den_out.at[wid, pl.ds(0, N)])

    plsc.subcore_barrier()

    pltpu.sync_copy(a128_sh.at[pl.ds(srow, N // NS)],
                    a128_out.at[cid, pl.ds(srow, N // NS)])
    pltpu.sync_copy(ae_sh.at[pl.ds(srow, N // NS)],
                    ae_out.at[cid, pl.ds(srow, N // NS)])


_edge_pass = pl.kernel(
    _edge_body,
    out_type=[
        jax.ShapeDtypeStruct((NC, NPAD, C), jnp.float32),
        jax.ShapeDtypeStruct((NC, NPAD, ED), jnp.float32),
        jax.ShapeDtypeStruct((NW, NPAD), jnp.float32),
    ],
    mesh=plsc.VectorSubcoreMesh(core_axis_name="c", subcore_axis_name="s",
                                num_cores=NC, num_subcores=NS),
    compiler_params=pltpu.CompilerParams(use_tc_tiling_on_sc=False,
                                         needs_layout_passes=False),
    scratch_types=[
        pltpu.VMEM((CH,), jnp.int32),
        pltpu.VMEM((CH,), jnp.int32),
        pltpu.VMEM((CH, C), jnp.float32),
        pltpu.VMEM((CH, C), jnp.float32),
        pltpu.VMEM((CH, ED), jnp.float32),
        pltpu.VMEM((CH, ED), jnp.float32),
        pltpu.VMEM((CH, ED), jnp.float32),
        pltpu.VMEM((16, 17), jnp.float32),
        pltpu.VMEM((CH,), jnp.float32),
        pltpu.VMEM((N,), jnp.float32),
        pltpu.VMEM_SHARED((N, C), jnp.float32),
        pltpu.VMEM_SHARED((N, ED), jnp.float32),
        pltpu.SemaphoreType.DMA,
        pltpu.SemaphoreType.DMA,
        pltpu.SemaphoreType.DMA,
        pltpu.SemaphoreType.DMA,
    ],
)


def _layer(x, src, dst, ea, z128, zae, zden, Wq, bq, Wk, bk, Wv, bv, We, Ws, bs):
    wcat = jnp.concatenate([Wq, Wk, Wv, Ws], axis=1)
    bcat = jnp.concatenate([bq, bk, bv, bs], axis=0)
    q, k, v, skip, qe = _proj(x, wcat, bcat, We.T)
    a128, ae, den = _edge_pass(src, dst, ea, q, k, v, qe, z128, zae, zden)
    return _combine(a128, ae, den, skip, We)


def kernel(x, pe, edge_index, edge_attr, batch,
           Wq0, bq0, Wk0, bk0, Wv0, bv0, We0, Ws0, bs0,
           Wq1, bq1, Wk1, bk1, Wv1, bv1, We1, Ws1, bs1,
           mlp_W1, mlp_b1, mlp_W2, mlp_b2):
    src = edge_index[0]
    dst = edge_index[1]
    xpad = jnp.zeros((NPAD, D), jnp.float32).at[:N].set(x)
    z128 = jnp.zeros((N, C), jnp.float32)
    zae = jnp.zeros((N, ED), jnp.float32)
    zden = jnp.zeros((N,), jnp.float32)

    h = _layer(xpad, src, dst, edge_attr, z128, zae, zden,
               Wq0, bq0, Wk0, bk0, Wv0, bv0, We0, Ws0, bs0)
    h = _layer(h, src, dst, edge_attr, z128, zae, zden,
               Wq1, bq1, Wk1, bk1, Wv1, bv1, We1, Ws1, bs1)
    h = _mm(h, mlp_W1, mlp_b1, act=True)
    h = _mm(h, mlp_W2, mlp_b2, act=False)
    return h[:N]
